# Initial kernel scaffold; baseline (speedup 1.0000x reference)
import jax
import jax.numpy as jnp
from jax.experimental import pallas as pl


def kernel(x, edge_index, edge_weight, W1p, b1p, W1n, b1n, W2p, b2p, W2n, b2n, g1, be1, g2, be2):
    raise NotImplementedError("write your pallas kernel here")


# speedup vs baseline: 8.3635x; 8.3635x over previous
import functools

import jax
import jax.numpy as jnp
from jax import lax
from jax.experimental import pallas as pl
from jax.experimental.pallas import tpu as pltpu
from jax.experimental.pallas import tpu_sc as plsc

NN = 10000
EE = 320000
L = 16
NC = 2
NS = 16
BATCH = 80
R = 20480
RPT = R // NS
DUMMY = 2 * NN
BN = 80
GRID_N = NN // BN
NEG_BLKS = NN // BN
CORE_BLKS = R // BN
CH1 = EE // (NC * NS)
CH2 = EE // NS
SB = 2000
NBS = SB // BATCH
NSUP1 = CH1 // SB
NSUP2 = CH2 // SB

_F32 = jnp.float32
_MESH = dict(core_axis_name="c", subcore_axis_name="s")


def _stage_and_index(t, e0, src_hbm, dst_hbm, w_hbm,
                     src_v, dst_v, w_v, gidx_v, sidx_v, goff):
    b0 = e0 + t * SB
    pltpu.sync_copy(src_hbm.at[pl.ds(b0, SB)], src_v)
    pltpu.sync_copy(dst_hbm.at[pl.ds(b0, SB)], dst_v)
    pltpu.sync_copy(w_hbm.at[pl.ds(b0, SB)], w_v)

    def idx_body(k, carry):
        b = k // (BATCH // L)
        j = k % (BATCH // L)
        off = k * L
        sv = src_v[pl.ds(off, L)]
        dv = dst_v[pl.ds(off, L)]
        wv = w_v[pl.ds(off, L)]
        neg = jnp.where(wv < 0.0, NN, 0)
        gidx_v[b, pl.ds(j * L, L)] = sv + neg + goff
        sidx_v[b, pl.ds(j * L, L)] = jnp.where(wv == 0.0, DUMMY, dv + neg)
        return carry

    lax.fori_loop(0, SB // L, idx_body, 0)


def _sc1_body(src_hbm, dst_hbm, w_hbm, tab_hbm, zrows_hbm, zcnt_hbm,
              psum_hbm, pcnt_hbm,
              src_v, dst_v, w_v, gidx_v, sidx_v, rows_v, ones_v,
              acc_sh, cnt_sh):
    c = lax.axis_index("c")
    s = lax.axis_index("s")
    wid = s * NC + c
    row0 = s * RPT
    pltpu.sync_copy(zrows_hbm.at[pl.ds(row0, RPT)], acc_sh.at[pl.ds(row0, RPT)])
    pltpu.sync_copy(zcnt_hbm.at[pl.ds(row0, RPT)], cnt_sh.at[pl.ds(row0, RPT)])
    for j in range(BATCH // L):
        ones_v[pl.ds(j * L, L)] = jnp.ones((L,), _F32)
    e0 = wid * CH1
    plsc.subcore_barrier()

    def super_body(t, carry):
        _stage_and_index(t, e0, src_hbm, dst_hbm, w_hbm,
                         src_v, dst_v, w_v, gidx_v, sidx_v, 0)

        def batch_body(k, carry2):
            pltpu.sync_copy(tab_hbm.at[gidx_v.at[k]], rows_v)
            pltpu.sync_copy(rows_v, acc_sh.at[sidx_v.at[k]], add=True)
            pltpu.sync_copy(ones_v, cnt_sh.at[sidx_v.at[k]], add=True)
            return carry2

        lax.fori_loop(0, NBS, batch_body, 0)
        return carry

    lax.fori_loop(0, NSUP1, super_body, 0)
    plsc.subcore_barrier()
    out0 = c * R + row0
    pltpu.sync_copy(acc_sh.at[pl.ds(row0, RPT)], psum_hbm.at[pl.ds(out0, RPT)])
    pltpu.sync_copy(cnt_sh.at[pl.ds(row0, RPT)], pcnt_hbm.at[pl.ds(out0, RPT)])


_sc1 = pl.kernel(
    _sc1_body,
    out_type=[
        jax.ShapeDtypeStruct((NC * R, 64), _F32),
        jax.ShapeDtypeStruct((NC * R,), _F32),
    ],
    mesh=plsc.VectorSubcoreMesh(**_MESH),
    compiler_params=pltpu.CompilerParams(use_tc_tiling_on_sc=False),
    scratch_types=[
        pltpu.VMEM((SB,), jnp.int32),
        pltpu.VMEM((SB,), jnp.int32),
        pltpu.VMEM((SB,), _F32),
        pltpu.VMEM((NBS, BATCH), jnp.int32),
        pltpu.VMEM((NBS, BATCH), jnp.int32),
        pltpu.VMEM((BATCH, 64), _F32),
        pltpu.VMEM((BATCH,), _F32),
        pltpu.VMEM_SHARED((R, 64), _F32),
        pltpu.VMEM_SHARED((R,), _F32),
    ],
)


def _sc2_body(tab_hbm, src_hbm, dst_hbm, w_hbm, zrows_hbm, acc2_hbm,
              src_v, dst_v, w_v, gidx_v, sidx_v, rows_v, acc_sh):
    c = lax.axis_index("c")
    s = lax.axis_index("s")
    row0 = s * RPT
    pltpu.sync_copy(zrows_hbm.at[pl.ds(row0, RPT)], acc_sh.at[pl.ds(row0, RPT)])
    e0 = s * CH2
    goff = c * (2 * NN)
    plsc.subcore_barrier()

    def super_body(t, carry):
        _stage_and_index(t, e0, src_hbm, dst_hbm, w_hbm,
                         src_v, dst_v, w_v, gidx_v, sidx_v, goff)

        def batch_body(k, carry2):
            pltpu.sync_copy(tab_hbm.at[gidx_v.at[k]], rows_v)
            pltpu.sync_copy(rows_v, acc_sh.at[sidx_v.at[k]], add=True)
            return carry2

        lax.fori_loop(0, NBS, batch_body, 0)
        return carry

    lax.fori_loop(0, NSUP2, super_body, 0)
    plsc.subcore_barrier()
    pltpu.sync_copy(acc_sh.at[pl.ds(row0, RPT)],
                    acc2_hbm.at[pl.ds(c * R + row0, RPT)])


_sc2 = pl.kernel(
    _sc2_body,
    out_type=[jax.ShapeDtypeStruct((NC * R, 64), _F32)],
    mesh=plsc.VectorSubcoreMesh(**_MESH),
    compiler_params=pltpu.CompilerParams(use_tc_tiling_on_sc=False),
    scratch_types=[
        pltpu.VMEM((SB,), jnp.int32),
        pltpu.VMEM((SB,), jnp.int32),
        pltpu.VMEM((SB,), _F32),
        pltpu.VMEM((NBS, BATCH), jnp.int32),
        pltpu.VMEM((NBS, BATCH), jnp.int32),
        pltpu.VMEM((BATCH, 64), _F32),
        pltpu.VMEM_SHARED((R, 64), _F32),
    ],
)


def _full(shape):
    nd = len(shape)
    return pl.BlockSpec(shape, lambda i: (0,) * nd)


def _tc1_body(x_ref, w1p_ref, w1n_ref, b1p_ref, b1n_ref, tab_ref, base_ref):
    xb = x_ref[...]
    w1p = w1p_ref[...]
    w1n = w1n_ref[...]
    tab_ref[0] = jnp.dot(xb, w1p[:128], preferred_element_type=_F32)
    tab_ref[1] = jnp.dot(xb, w1n[:128], preferred_element_type=_F32)
    base_ref[0] = jnp.dot(xb, w1p[128:], preferred_element_type=_F32) + b1p_ref[...]
    base_ref[1] = jnp.dot(xb, w1n[128:], preferred_element_type=_F32) + b1n_ref[...]


_tc1 = pl.pallas_call(
    _tc1_body,
    grid=(GRID_N,),
    in_specs=[
        pl.BlockSpec((BN, 128), lambda i: (i, 0)),
        _full((256, 64)), _full((256, 64)), _full((64,)), _full((64,)),
    ],
    out_specs=[
        pl.BlockSpec((2, BN, 64), lambda i: (0, i, 0)),
        pl.BlockSpec((2, BN, 64), lambda i: (0, i, 0)),
    ],
    out_shape=[
        jax.ShapeDtypeStruct((2, NN, 64), _F32),
        jax.ShapeDtypeStruct((2, NN, 64), _F32),
    ],
)


def _ln_halves(hp, hn, g_ref, be_ref):
    m = (jnp.sum(hp, 1, keepdims=True) + jnp.sum(hn, 1, keepdims=True)) / 128.0
    dp = hp - m
    dn = hn - m
    var = (jnp.sum(dp * dp, 1, keepdims=True)
           + jnp.sum(dn * dn, 1, keepdims=True)) / 128.0
    inv = lax.rsqrt(var + 1e-5)
    g = g_ref[...]
    be = be_ref[...]
    return dp * inv * g[:64] + be[:64], dn * inv * g[64:] + be[64:]


def _tc2_body(sp0, sp1, sn0, sn1, cp0, cp1, cn0, cn1, base1_ref,
              w2p_ref, w2n_ref, b2p_ref, b2n_ref, g1_ref, be1_ref,
              t2_ref, base2_ref):
    cp = jnp.maximum(cp0[...] + cp1[...], 1.0)
    cn = jnp.maximum(cn0[...] + cn1[...], 1.0)
    hp = (sp0[...] + sp1[...]) / cp + base1_ref[0]
    hn = (sn0[...] + sn1[...]) / cn + base1_ref[1]
    hp = jnp.maximum(hp, 0.0)
    hn = jnp.maximum(hn, 0.0)
    hp, hn = _ln_halves(hp, hn, g1_ref, be1_ref)
    w2p = w2p_ref[...]
    w2n = w2n_ref[...]
    t2_ref[0, 0] = jnp.dot(hp, w2p[:64], preferred_element_type=_F32)
    t2_ref[0, 1] = jnp.dot(hn, w2p[64:128], preferred_element_type=_F32)
    t2_ref[1, 0] = jnp.dot(hn, w2n[:64], preferred_element_type=_F32)
    t2_ref[1, 1] = jnp.dot(hp, w2n[64:128], preferred_element_type=_F32)
    base2_ref[0] = jnp.dot(hp, w2p[128:], preferred_element_type=_F32) + b2p_ref[...]
    base2_ref[1] = jnp.dot(hn, w2n[128:], preferred_element_type=_F32) + b2n_ref[...]


def _psum_specs():
    return [
        pl.BlockSpec((BN, 64), lambda i: (i, 0)),
        pl.BlockSpec((BN, 64), lambda i: (i + CORE_BLKS, 0)),
        pl.BlockSpec((BN, 64), lambda i: (i + NEG_BLKS, 0)),
        pl.BlockSpec((BN, 64), lambda i: (i + CORE_BLKS + NEG_BLKS, 0)),
    ]


def _pcnt_specs():
    return [
        pl.BlockSpec((BN, 1), lambda i: (i, 0)),
        pl.BlockSpec((BN, 1), lambda i: (i + CORE_BLKS, 0)),
        pl.BlockSpec((BN, 1), lambda i: (i + NEG_BLKS, 0)),
        pl.BlockSpec((BN, 1), lambda i: (i + CORE_BLKS + NEG_BLKS, 0)),
    ]


def _tc2_specs():
    ps = _psum_specs()
    cs = _pcnt_specs()
    return [ps[0], ps[1], ps[2], ps[3], cs[0], cs[1], cs[2], cs[3],
            pl.BlockSpec((2, BN, 64), lambda i: (0, i, 0)),
            _full((192, 64)), _full((192, 64)), _full((64,)), _full((64,)),
            _full((128,)), _full((128,))]


_tc2 = pl.pallas_call(
    _tc2_body,
    grid=(GRID_N,),
    in_specs=_tc2_specs(),
    out_specs=[
        pl.BlockSpec((2, 2, BN, 64), lambda i: (0, 0, i, 0)),
        pl.BlockSpec((2, BN, 64), lambda i: (0, i, 0)),
    ],
    out_shape=[
        jax.ShapeDtypeStruct((2, 2, NN, 64), _F32),
        jax.ShapeDtypeStruct((2, NN, 64), _F32),
    ],
)


def _tc3_body(ap, an, bp, bn, cp0, cp1, cn0, cn1, base2_ref,
              g2_ref, be2_ref, out_ref):
    invp = 1.0 / jnp.maximum(cp0[...] + cp1[...], 1.0)
    invn = 1.0 / jnp.maximum(cn0[...] + cn1[...], 1.0)
    op = ap[...] * invp + an[...] * invn + base2_ref[0]
    on = bp[...] * invp + bn[...] * invn + base2_ref[1]
    op = jnp.maximum(op, 0.0)
    on = jnp.maximum(on, 0.0)
    op, on = _ln_halves(op, on, g2_ref, be2_ref)
    out_ref[:, :64] = op
    out_ref[:, 64:] = on


def _acc2_specs():
    return [
        pl.BlockSpec((BN, 64), lambda i: (i, 0)),
        pl.BlockSpec((BN, 64), lambda i: (i + NEG_BLKS, 0)),
        pl.BlockSpec((BN, 64), lambda i: (i + CORE_BLKS, 0)),
        pl.BlockSpec((BN, 64), lambda i: (i + CORE_BLKS + NEG_BLKS, 0)),
    ]


_tc3 = pl.pallas_call(
    _tc3_body,
    grid=(GRID_N,),
    in_specs=_acc2_specs() + _pcnt_specs() + [
        pl.BlockSpec((2, BN, 64), lambda i: (0, i, 0)),
        _full((128,)), _full((128,)),
    ],
    out_specs=pl.BlockSpec((BN, 128), lambda i: (i, 0)),
    out_shape=jax.ShapeDtypeStruct((NN, 128), _F32),
)


def kernel(x, edge_index, edge_weight, W1p, b1p, W1n, b1n,
           W2p, b2p, W2n, b2n, g1, be1, g2, be2):
    src = edge_index[0]
    dst = edge_index[1]
    zrows = jnp.zeros((R, 64), _F32)
    zcnt = jnp.zeros((R,), _F32)

    tab1, base1 = _tc1(x, W1p, W1n, b1p, b1n)
    psum, pcnt = _sc1(src, dst, edge_weight,
                      tab1.reshape(2 * NN, 64), zrows, zcnt)
    pcnt2 = pcnt.reshape(NC * R, 1)
    t2, base2 = _tc2(psum, psum, psum, psum, pcnt2, pcnt2, pcnt2, pcnt2,
                     base1, W2p, W2n, b2p, b2n, g1, be1)
    acc2 = _sc2(t2.reshape(4 * NN, 64), src, dst, edge_weight, zrows)[0]
    out = _tc3(acc2, acc2, acc2, acc2, pcnt2, pcnt2, pcnt2, pcnt2,
               base2, g2, be2)
    return out


# speedup vs baseline: 9.4450x; 1.1293x over previous
import functools

import jax
import jax.numpy as jnp
from jax import lax
from jax.experimental import pallas as pl
from jax.experimental.pallas import tpu as pltpu
from jax.experimental.pallas import tpu_sc as plsc

NN = 10000
EE = 320000
L = 16
NC = 2
NS = 16
BATCH = 80
R = 20480
RPT = R // NS
DUMMY = 2 * NN
BN = 80
GRID_N = NN // BN
NEG_BLKS = NN // BN
CORE_BLKS = R // BN
CH1 = EE // (NC * NS)
CH2 = EE // NS
SB = 2000
NBS = SB // BATCH
NSUP1 = CH1 // SB
NSUP2 = CH2 // SB

_F32 = jnp.float32
_MESH = dict(core_axis_name="c", subcore_axis_name="s")


def _stage_and_index(t, e0, src_hbm, dst_hbm, w_hbm,
                     src_v, dst_v, w_v, gidx_v, sidx_v, goff):
    b0 = e0 + t * SB
    pltpu.sync_copy(src_hbm.at[pl.ds(b0, SB)], src_v)
    pltpu.sync_copy(dst_hbm.at[pl.ds(b0, SB)], dst_v)
    pltpu.sync_copy(w_hbm.at[pl.ds(b0, SB)], w_v)

    def idx_body(k, carry):
        b = k // (BATCH // L)
        j = k % (BATCH // L)
        off = k * L
        sv = src_v[pl.ds(off, L)]
        dv = dst_v[pl.ds(off, L)]
        wv = w_v[pl.ds(off, L)]
        neg = jnp.where(wv < 0.0, NN, 0)
        gidx_v[b, pl.ds(j * L, L)] = sv + neg + goff
        sidx_v[b, pl.ds(j * L, L)] = jnp.where(wv == 0.0, DUMMY, dv + neg)
        return carry

    lax.fori_loop(0, SB // L, idx_body, 0)


def _stream_batches(tab_hbm, gidx_v, sidx_v, rows0, rows1, sem0, sem1,
                    acc_sh, scatter_extra):
    pltpu.async_copy(tab_hbm.at[gidx_v.at[0]], rows0, sem0)

    def pair_body(kk, carry):
        k0 = 2 * kk
        pltpu.make_async_copy(tab_hbm.at[gidx_v.at[k0]], rows0, sem0).wait()
        pltpu.async_copy(tab_hbm.at[gidx_v.at[k0 + 1]], rows1, sem1)
        pltpu.sync_copy(rows0, acc_sh.at[sidx_v.at[k0]], add=True)
        scatter_extra(k0)
        pltpu.make_async_copy(tab_hbm.at[gidx_v.at[k0 + 1]], rows1, sem1).wait()
        pltpu.async_copy(tab_hbm.at[gidx_v.at[k0 + 2]], rows0, sem0)
        pltpu.sync_copy(rows1, acc_sh.at[sidx_v.at[k0 + 1]], add=True)
        scatter_extra(k0 + 1)
        return carry

    lax.fori_loop(0, (NBS - 1) // 2, pair_body, 0)
    last = NBS - 1
    pltpu.make_async_copy(tab_hbm.at[gidx_v.at[last]], rows0, sem0).wait()
    pltpu.sync_copy(rows0, acc_sh.at[sidx_v.at[last]], add=True)
    scatter_extra(last)


def _sc1_body(src_hbm, dst_hbm, w_hbm, tab_hbm, zrows_hbm, zcnt_hbm,
              psum_hbm, pcnt_hbm,
              src_v, dst_v, w_v, gidx_v, sidx_v, rows0_v, rows1_v, ones_v,
              acc_sh, cnt_sh, sem0, sem1):
    c = lax.axis_index("c")
    s = lax.axis_index("s")
    wid = s * NC + c
    row0 = s * RPT
    pltpu.sync_copy(zrows_hbm.at[pl.ds(row0, RPT)], acc_sh.at[pl.ds(row0, RPT)])
    pltpu.sync_copy(zcnt_hbm.at[pl.ds(row0, RPT)], cnt_sh.at[pl.ds(row0, RPT)])
    for j in range(BATCH // L):
        ones_v[pl.ds(j * L, L)] = jnp.ones((L,), _F32)
    e0 = wid * CH1
    plsc.subcore_barrier()

    def super_body(t, carry):
        _stage_and_index(t, e0, src_hbm, dst_hbm, w_hbm,
                         src_v, dst_v, w_v, gidx_v, sidx_v, 0)

        def cnt_scatter(k):
            pltpu.sync_copy(ones_v, cnt_sh.at[sidx_v.at[k]], add=True)

        _stream_batches(tab_hbm, gidx_v, sidx_v, rows0_v, rows1_v,
                        sem0, sem1, acc_sh, cnt_scatter)
        return carry

    lax.fori_loop(0, NSUP1, super_body, 0)
    plsc.subcore_barrier()
    out0 = c * R + row0
    pltpu.sync_copy(acc_sh.at[pl.ds(row0, RPT)], psum_hbm.at[pl.ds(out0, RPT)])
    pltpu.sync_copy(cnt_sh.at[pl.ds(row0, RPT)], pcnt_hbm.at[pl.ds(out0, RPT)])


_sc1 = pl.kernel(
    _sc1_body,
    out_type=[
        jax.ShapeDtypeStruct((NC * R, 64), _F32),
        jax.ShapeDtypeStruct((NC * R,), _F32),
    ],
    mesh=plsc.VectorSubcoreMesh(**_MESH),
    compiler_params=pltpu.CompilerParams(use_tc_tiling_on_sc=False),
    scratch_types=[
        pltpu.VMEM((SB,), jnp.int32),
        pltpu.VMEM((SB,), jnp.int32),
        pltpu.VMEM((SB,), _F32),
        pltpu.VMEM((NBS, BATCH), jnp.int32),
        pltpu.VMEM((NBS, BATCH), jnp.int32),
        pltpu.VMEM((BATCH, 64), _F32),
        pltpu.VMEM((BATCH, 64), _F32),
        pltpu.VMEM((BATCH,), _F32),
        pltpu.VMEM_SHARED((R, 64), _F32),
        pltpu.VMEM_SHARED((R,), _F32),
        pltpu.SemaphoreType.DMA,
        pltpu.SemaphoreType.DMA,
    ],
)


def _sc2_body(tab_hbm, src_hbm, dst_hbm, w_hbm, zrows_hbm, acc2_hbm,
              src_v, dst_v, w_v, gidx_v, sidx_v, rows0_v, rows1_v,
              acc_sh, sem0, sem1):
    c = lax.axis_index("c")
    s = lax.axis_index("s")
    row0 = s * RPT
    pltpu.sync_copy(zrows_hbm.at[pl.ds(row0, RPT)], acc_sh.at[pl.ds(row0, RPT)])
    e0 = s * CH2
    goff = c * (2 * NN)
    plsc.subcore_barrier()

    def super_body(t, carry):
        _stage_and_index(t, e0, src_hbm, dst_hbm, w_hbm,
                         src_v, dst_v, w_v, gidx_v, sidx_v, goff)
        _stream_batches(tab_hbm, gidx_v, sidx_v, rows0_v, rows1_v,
                        sem0, sem1, acc_sh, lambda k: None)
        return carry

    lax.fori_loop(0, NSUP2, super_body, 0)
    plsc.subcore_barrier()
    pltpu.sync_copy(acc_sh.at[pl.ds(row0, RPT)],
                    acc2_hbm.at[pl.ds(c * R + row0, RPT)])


_sc2 = pl.kernel(
    _sc2_body,
    out_type=[jax.ShapeDtypeStruct((NC * R, 64), _F32)],
    mesh=plsc.VectorSubcoreMesh(**_MESH),
    compiler_params=pltpu.CompilerParams(use_tc_tiling_on_sc=False),
    scratch_types=[
        pltpu.VMEM((SB,), jnp.int32),
        pltpu.VMEM((SB,), jnp.int32),
        pltpu.VMEM((SB,), _F32),
        pltpu.VMEM((NBS, BATCH), jnp.int32),
        pltpu.VMEM((NBS, BATCH), jnp.int32),
        pltpu.VMEM((BATCH, 64), _F32),
        pltpu.VMEM((BATCH, 64), _F32),
        pltpu.VMEM_SHARED((R, 64), _F32),
        pltpu.SemaphoreType.DMA,
        pltpu.SemaphoreType.DMA,
    ],
)


def _full(shape):
    nd = len(shape)
    return pl.BlockSpec(shape, lambda i: (0,) * nd)


def _tc1_body(x_ref, w1p_ref, w1n_ref, b1p_ref, b1n_ref, tab_ref, base_ref):
    xb = x_ref[...]
    w1p = w1p_ref[...]
    w1n = w1n_ref[...]
    tab_ref[0] = jnp.dot(xb, w1p[:128], preferred_element_type=_F32)
    tab_ref[1] = jnp.dot(xb, w1n[:128], preferred_element_type=_F32)
    base_ref[0] = jnp.dot(xb, w1p[128:], preferred_element_type=_F32) + b1p_ref[...]
    base_ref[1] = jnp.dot(xb, w1n[128:], preferred_element_type=_F32) + b1n_ref[...]


_tc1 = pl.pallas_call(
    _tc1_body,
    grid=(GRID_N,),
    in_specs=[
        pl.BlockSpec((BN, 128), lambda i: (i, 0)),
        _full((256, 64)), _full((256, 64)), _full((64,)), _full((64,)),
    ],
    out_specs=[
        pl.BlockSpec((2, BN, 64), lambda i: (0, i, 0)),
        pl.BlockSpec((2, BN, 64), lambda i: (0, i, 0)),
    ],
    out_shape=[
        jax.ShapeDtypeStruct((2, NN, 64), _F32),
        jax.ShapeDtypeStruct((2, NN, 64), _F32),
    ],
)


def _ln_halves(hp, hn, g_ref, be_ref):
    m = (jnp.sum(hp, 1, keepdims=True) + jnp.sum(hn, 1, keepdims=True)) / 128.0
    dp = hp - m
    dn = hn - m
    var = (jnp.sum(dp * dp, 1, keepdims=True)
           + jnp.sum(dn * dn, 1, keepdims=True)) / 128.0
    inv = lax.rsqrt(var + 1e-5)
    g = g_ref[...]
    be = be_ref[...]
    return dp * inv * g[:64] + be[:64], dn * inv * g[64:] + be[64:]


def _tc2_body(sp0, sp1, sn0, sn1, cp0, cp1, cn0, cn1, base1_ref,
              w2p_ref, w2n_ref, b2p_ref, b2n_ref, g1_ref, be1_ref,
              t2_ref, base2_ref):
    cp = jnp.maximum(cp0[...] + cp1[...], 1.0)
    cn = jnp.maximum(cn0[...] + cn1[...], 1.0)
    hp = (sp0[...] + sp1[...]) / cp + base1_ref[0]
    hn = (sn0[...] + sn1[...]) / cn + base1_ref[1]
    hp = jnp.maximum(hp, 0.0)
    hn = jnp.maximum(hn, 0.0)
    hp, hn = _ln_halves(hp, hn, g1_ref, be1_ref)
    w2p = w2p_ref[...]
    w2n = w2n_ref[...]
    t2_ref[0, 0] = jnp.dot(hp, w2p[:64], preferred_element_type=_F32)
    t2_ref[0, 1] = jnp.dot(hn, w2p[64:128], preferred_element_type=_F32)
    t2_ref[1, 0] = jnp.dot(hn, w2n[:64], preferred_element_type=_F32)
    t2_ref[1, 1] = jnp.dot(hp, w2n[64:128], preferred_element_type=_F32)
    base2_ref[0] = jnp.dot(hp, w2p[128:], preferred_element_type=_F32) + b2p_ref[...]
    base2_ref[1] = jnp.dot(hn, w2n[128:], preferred_element_type=_F32) + b2n_ref[...]


def _psum_specs():
    return [
        pl.BlockSpec((BN, 64), lambda i: (i, 0)),
        pl.BlockSpec((BN, 64), lambda i: (i + CORE_BLKS, 0)),
        pl.BlockSpec((BN, 64), lambda i: (i + NEG_BLKS, 0)),
        pl.BlockSpec((BN, 64), lambda i: (i + CORE_BLKS + NEG_BLKS, 0)),
    ]


def _pcnt_specs():
    return [
        pl.BlockSpec((BN, 1), lambda i: (i, 0)),
        pl.BlockSpec((BN, 1), lambda i: (i + CORE_BLKS, 0)),
        pl.BlockSpec((BN, 1), lambda i: (i + NEG_BLKS, 0)),
        pl.BlockSpec((BN, 1), lambda i: (i + CORE_BLKS + NEG_BLKS, 0)),
    ]


def _tc2_specs():
    ps = _psum_specs()
    cs = _pcnt_specs()
    return [ps[0], ps[1], ps[2], ps[3], cs[0], cs[1], cs[2], cs[3],
            pl.BlockSpec((2, BN, 64), lambda i: (0, i, 0)),
            _full((192, 64)), _full((192, 64)), _full((64,)), _full((64,)),
            _full((128,)), _full((128,))]


_tc2 = pl.pallas_call(
    _tc2_body,
    grid=(GRID_N,),
    in_specs=_tc2_specs(),
    out_specs=[
        pl.BlockSpec((2, 2, BN, 64), lambda i: (0, 0, i, 0)),
        pl.BlockSpec((2, BN, 64), lambda i: (0, i, 0)),
    ],
    out_shape=[
        jax.ShapeDtypeStruct((2, 2, NN, 64), _F32),
        jax.ShapeDtypeStruct((2, NN, 64), _F32),
    ],
)


def _tc3_body(ap, an, bp, bn, cp0, cp1, cn0, cn1, base2_ref,
              g2_ref, be2_ref, out_ref):
    invp = 1.0 / jnp.maximum(cp0[...] + cp1[...], 1.0)
    invn = 1.0 / jnp.maximum(cn0[...] + cn1[...], 1.0)
    op = ap[...] * invp + an[...] * invn + base2_ref[0]
    on = bp[...] * invp + bn[...] * invn + base2_ref[1]
    op = jnp.maximum(op, 0.0)
    on = jnp.maximum(on, 0.0)
    op, on = _ln_halves(op, on, g2_ref, be2_ref)
    out_ref[:, :64] = op
    out_ref[:, 64:] = on


def _acc2_specs():
    return [
        pl.BlockSpec((BN, 64), lambda i: (i, 0)),
        pl.BlockSpec((BN, 64), lambda i: (i + NEG_BLKS, 0)),
        pl.BlockSpec((BN, 64), lambda i: (i + CORE_BLKS, 0)),
        pl.BlockSpec((BN, 64), lambda i: (i + CORE_BLKS + NEG_BLKS, 0)),
    ]


_tc3 = pl.pallas_call(
    _tc3_body,
    grid=(GRID_N,),
    in_specs=_acc2_specs() + _pcnt_specs() + [
        pl.BlockSpec((2, BN, 64), lambda i: (0, i, 0)),
        _full((128,)), _full((128,)),
    ],
    out_specs=pl.BlockSpec((BN, 128), lambda i: (i, 0)),
    out_shape=jax.ShapeDtypeStruct((NN, 128), _F32),
)


def kernel(x, edge_index, edge_weight, W1p, b1p, W1n, b1n,
           W2p, b2p, W2n, b2n, g1, be1, g2, be2):
    src = edge_index[0]
    dst = edge_index[1]
    zrows = jnp.zeros((R, 64), _F32)
    zcnt = jnp.zeros((R,), _F32)

    tab1, base1 = _tc1(x, W1p, W1n, b1p, b1n)
    psum, pcnt = _sc1(src, dst, edge_weight,
                      tab1.reshape(2 * NN, 64), zrows, zcnt)
    pcnt2 = pcnt.reshape(NC * R, 1)
    t2, base2 = _tc2(psum, psum, psum, psum, pcnt2, pcnt2, pcnt2, pcnt2,
                     base1, W2p, W2n, b2p, b2n, g1, be1)
    acc2 = _sc2(t2.reshape(4 * NN, 64), src, dst, edge_weight, zrows)[0]
    out = _tc3(acc2, acc2, acc2, acc2, pcnt2, pcnt2, pcnt2, pcnt2,
               base2, g2, be2)
    return out


# speedup vs baseline: 11.9742x; 1.2678x over previous
import functools

import jax
import jax.numpy as jnp
from jax import lax
from jax.experimental import pallas as pl
from jax.experimental.pallas import tpu as pltpu
from jax.experimental.pallas import tpu_sc as plsc

NN = 10000
EE = 320000
L = 16
NC = 2
NS = 16
BATCH = 80
R = 25600
RPT = R // NS
DUMMY = 2 * NN
BN = 400
GRID_N = NN // BN
NEG_BLKS = NN // BN
CORE_BLKS = R // BN
CH1 = EE // (NC * NS)
CH2 = EE // NS
SB = 2000
NBS = SB // BATCH
NSUP1 = CH1 // SB
NSUP2 = CH2 // SB

_F32 = jnp.float32
_MESH = dict(core_axis_name="c", subcore_axis_name="s")


def _stage_and_index(t, e0, src_hbm, dst_hbm, w_hbm,
                     src_v, dst_v, w_v, gidx_v, sidx_v, goff):
    b0 = e0 + t * SB
    pltpu.sync_copy(src_hbm.at[pl.ds(b0, SB)], src_v)
    pltpu.sync_copy(dst_hbm.at[pl.ds(b0, SB)], dst_v)
    pltpu.sync_copy(w_hbm.at[pl.ds(b0, SB)], w_v)

    def idx_body(k, carry):
        b = k // (BATCH // L)
        j = k % (BATCH // L)
        off = k * L
        sv = src_v[pl.ds(off, L)]
        dv = dst_v[pl.ds(off, L)]
        wv = w_v[pl.ds(off, L)]
        neg = jnp.where(wv < 0.0, NN, 0)
        gidx_v[b, pl.ds(j * L, L)] = sv + neg + goff
        sidx_v[b, pl.ds(j * L, L)] = jnp.where(wv == 0.0, DUMMY, dv + neg)
        return carry

    lax.fori_loop(0, SB // L, idx_body, 0)


def _stream_batches(tab_hbm, gidx_v, sidx_v, rows0, rows1, sem0, sem1,
                    acc_sh, scatter_extra):
    pltpu.async_copy(tab_hbm.at[gidx_v.at[0]], rows0, sem0)

    def pair_body(kk, carry):
        k0 = 2 * kk
        pltpu.make_async_copy(tab_hbm.at[gidx_v.at[k0]], rows0, sem0).wait()
        pltpu.async_copy(tab_hbm.at[gidx_v.at[k0 + 1]], rows1, sem1)
        pltpu.sync_copy(rows0, acc_sh.at[sidx_v.at[k0]], add=True)
        scatter_extra(k0)
        pltpu.make_async_copy(tab_hbm.at[gidx_v.at[k0 + 1]], rows1, sem1).wait()
        pltpu.async_copy(tab_hbm.at[gidx_v.at[k0 + 2]], rows0, sem0)
        pltpu.sync_copy(rows1, acc_sh.at[sidx_v.at[k0 + 1]], add=True)
        scatter_extra(k0 + 1)
        return carry

    lax.fori_loop(0, (NBS - 1) // 2, pair_body, 0)
    last = NBS - 1
    pltpu.make_async_copy(tab_hbm.at[gidx_v.at[last]], rows0, sem0).wait()
    pltpu.sync_copy(rows0, acc_sh.at[sidx_v.at[last]], add=True)
    scatter_extra(last)


def _sc1_body(src_hbm, dst_hbm, w_hbm, tab_hbm, zrows_hbm, zcnt_hbm,
              psum_hbm, pcnt_hbm,
              src_v, dst_v, w_v, gidx_v, sidx_v, rows0_v, rows1_v, ones_v,
              acc_sh, cnt_sh, sem0, sem1):
    c = lax.axis_index("c")
    s = lax.axis_index("s")
    wid = s * NC + c
    row0 = s * RPT
    pltpu.sync_copy(zrows_hbm.at[pl.ds(row0, RPT)], acc_sh.at[pl.ds(row0, RPT)])
    pltpu.sync_copy(zcnt_hbm.at[pl.ds(row0, RPT)], cnt_sh.at[pl.ds(row0, RPT)])
    for j in range(BATCH // L):
        ones_v[pl.ds(j * L, L)] = jnp.ones((L,), _F32)
    e0 = wid * CH1
    plsc.subcore_barrier()

    def super_body(t, carry):
        _stage_and_index(t, e0, src_hbm, dst_hbm, w_hbm,
                         src_v, dst_v, w_v, gidx_v, sidx_v, 0)

        def cnt_scatter(k):
            pltpu.sync_copy(ones_v, cnt_sh.at[sidx_v.at[k]], add=True)

        _stream_batches(tab_hbm, gidx_v, sidx_v, rows0_v, rows1_v,
                        sem0, sem1, acc_sh, cnt_scatter)
        return carry

    lax.fori_loop(0, NSUP1, super_body, 0)
    plsc.subcore_barrier()
    out0 = c * R + row0
    pltpu.sync_copy(acc_sh.at[pl.ds(row0, RPT)], psum_hbm.at[pl.ds(out0, RPT)])
    pltpu.sync_copy(cnt_sh.at[pl.ds(row0, RPT)], pcnt_hbm.at[pl.ds(out0, RPT)])


_sc1 = pl.kernel(
    _sc1_body,
    out_type=[
        jax.ShapeDtypeStruct((NC * R, 64), _F32),
        jax.ShapeDtypeStruct((NC * R,), _F32),
    ],
    mesh=plsc.VectorSubcoreMesh(**_MESH),
    compiler_params=pltpu.CompilerParams(use_tc_tiling_on_sc=False),
    scratch_types=[
        pltpu.VMEM((SB,), jnp.int32),
        pltpu.VMEM((SB,), jnp.int32),
        pltpu.VMEM((SB,), _F32),
        pltpu.VMEM((NBS, BATCH), jnp.int32),
        pltpu.VMEM((NBS, BATCH), jnp.int32),
        pltpu.VMEM((BATCH, 64), _F32),
        pltpu.VMEM((BATCH, 64), _F32),
        pltpu.VMEM((BATCH,), _F32),
        pltpu.VMEM_SHARED((R, 64), _F32),
        pltpu.VMEM_SHARED((R,), _F32),
        pltpu.SemaphoreType.DMA,
        pltpu.SemaphoreType.DMA,
    ],
)


def _sc2_body(tab_hbm, src_hbm, dst_hbm, w_hbm, zrows_hbm, acc2_hbm,
              src_v, dst_v, w_v, gidx_v, sidx_v, rows0_v, rows1_v,
              acc_sh, sem0, sem1):
    c = lax.axis_index("c")
    s = lax.axis_index("s")
    row0 = s * RPT
    pltpu.sync_copy(zrows_hbm.at[pl.ds(row0, RPT)], acc_sh.at[pl.ds(row0, RPT)])
    e0 = s * CH2
    goff = c * (2 * NN)
    plsc.subcore_barrier()

    def super_body(t, carry):
        _stage_and_index(t, e0, src_hbm, dst_hbm, w_hbm,
                         src_v, dst_v, w_v, gidx_v, sidx_v, goff)
        _stream_batches(tab_hbm, gidx_v, sidx_v, rows0_v, rows1_v,
                        sem0, sem1, acc_sh, lambda k: None)
        return carry

    lax.fori_loop(0, NSUP2, super_body, 0)
    plsc.subcore_barrier()
    pltpu.sync_copy(acc_sh.at[pl.ds(row0, RPT)],
                    acc2_hbm.at[pl.ds(c * R + row0, RPT)])


_sc2 = pl.kernel(
    _sc2_body,
    out_type=[jax.ShapeDtypeStruct((NC * R, 64), _F32)],
    mesh=plsc.VectorSubcoreMesh(**_MESH),
    compiler_params=pltpu.CompilerParams(use_tc_tiling_on_sc=False),
    scratch_types=[
        pltpu.VMEM((SB,), jnp.int32),
        pltpu.VMEM((SB,), jnp.int32),
        pltpu.VMEM((SB,), _F32),
        pltpu.VMEM((NBS, BATCH), jnp.int32),
        pltpu.VMEM((NBS, BATCH), jnp.int32),
        pltpu.VMEM((BATCH, 64), _F32),
        pltpu.VMEM((BATCH, 64), _F32),
        pltpu.VMEM_SHARED((R, 64), _F32),
        pltpu.SemaphoreType.DMA,
        pltpu.SemaphoreType.DMA,
    ],
)


def _full(shape):
    nd = len(shape)
    return pl.BlockSpec(shape, lambda i: (0,) * nd)


def _tc1_body(x_ref, w1p_ref, w1n_ref, b1p_ref, b1n_ref, tab_ref, base_ref):
    xb = x_ref[...]
    w1p = w1p_ref[...]
    w1n = w1n_ref[...]
    tab_ref[0] = jnp.dot(xb, w1p[:128], preferred_element_type=_F32)
    tab_ref[1] = jnp.dot(xb, w1n[:128], preferred_element_type=_F32)
    base_ref[0] = jnp.dot(xb, w1p[128:], preferred_element_type=_F32) + b1p_ref[...]
    base_ref[1] = jnp.dot(xb, w1n[128:], preferred_element_type=_F32) + b1n_ref[...]


_tc1 = pl.pallas_call(
    _tc1_body,
    grid=(GRID_N,),
    in_specs=[
        pl.BlockSpec((BN, 128), lambda i: (i, 0)),
        _full((256, 64)), _full((256, 64)), _full((64,)), _full((64,)),
    ],
    out_specs=[
        pl.BlockSpec((2, BN, 64), lambda i: (0, i, 0)),
        pl.BlockSpec((2, BN, 64), lambda i: (0, i, 0)),
    ],
    out_shape=[
        jax.ShapeDtypeStruct((2, NN, 64), _F32),
        jax.ShapeDtypeStruct((2, NN, 64), _F32),
    ],
)


def _ln_halves(hp, hn, g_ref, be_ref):
    m = (jnp.sum(hp, 1, keepdims=True) + jnp.sum(hn, 1, keepdims=True)) / 128.0
    dp = hp - m
    dn = hn - m
    var = (jnp.sum(dp * dp, 1, keepdims=True)
           + jnp.sum(dn * dn, 1, keepdims=True)) / 128.0
    inv = lax.rsqrt(var + 1e-5)
    g = g_ref[...]
    be = be_ref[...]
    return dp * inv * g[:64] + be[:64], dn * inv * g[64:] + be[64:]


def _tc2_body(sp0, sp1, sn0, sn1, cp0, cp1, cn0, cn1, base1_ref,
              w2p_ref, w2n_ref, b2p_ref, b2n_ref, g1_ref, be1_ref,
              t2_ref, base2_ref):
    cp = jnp.maximum(cp0[...] + cp1[...], 1.0)
    cn = jnp.maximum(cn0[...] + cn1[...], 1.0)
    hp = (sp0[...] + sp1[...]) / cp + base1_ref[0]
    hn = (sn0[...] + sn1[...]) / cn + base1_ref[1]
    hp = jnp.maximum(hp, 0.0)
    hn = jnp.maximum(hn, 0.0)
    hp, hn = _ln_halves(hp, hn, g1_ref, be1_ref)
    w2p = w2p_ref[...]
    w2n = w2n_ref[...]
    t2_ref[0, 0] = jnp.dot(hp, w2p[:64], preferred_element_type=_F32)
    t2_ref[0, 1] = jnp.dot(hn, w2p[64:128], preferred_element_type=_F32)
    t2_ref[1, 0] = jnp.dot(hn, w2n[:64], preferred_element_type=_F32)
    t2_ref[1, 1] = jnp.dot(hp, w2n[64:128], preferred_element_type=_F32)
    base2_ref[0] = jnp.dot(hp, w2p[128:], preferred_element_type=_F32) + b2p_ref[...]
    base2_ref[1] = jnp.dot(hn, w2n[128:], preferred_element_type=_F32) + b2n_ref[...]


def _psum_specs():
    return [
        pl.BlockSpec((BN, 64), lambda i: (i, 0)),
        pl.BlockSpec((BN, 64), lambda i: (i + CORE_BLKS, 0)),
        pl.BlockSpec((BN, 64), lambda i: (i + NEG_BLKS, 0)),
        pl.BlockSpec((BN, 64), lambda i: (i + CORE_BLKS + NEG_BLKS, 0)),
    ]


def _pcnt_specs():
    return [
        pl.BlockSpec((BN, 1), lambda i: (i, 0)),
        pl.BlockSpec((BN, 1), lambda i: (i + CORE_BLKS, 0)),
        pl.BlockSpec((BN, 1), lambda i: (i + NEG_BLKS, 0)),
        pl.BlockSpec((BN, 1), lambda i: (i + CORE_BLKS + NEG_BLKS, 0)),
    ]


def _tc2_specs():
    ps = _psum_specs()
    cs = _pcnt_specs()
    return [ps[0], ps[1], ps[2], ps[3], cs[0], cs[1], cs[2], cs[3],
            pl.BlockSpec((2, BN, 64), lambda i: (0, i, 0)),
            _full((192, 64)), _full((192, 64)), _full((64,)), _full((64,)),
            _full((128,)), _full((128,))]


_tc2 = pl.pallas_call(
    _tc2_body,
    grid=(GRID_N,),
    in_specs=_tc2_specs(),
    out_specs=[
        pl.BlockSpec((2, 2, BN, 64), lambda i: (0, 0, i, 0)),
        pl.BlockSpec((2, BN, 64), lambda i: (0, i, 0)),
    ],
    out_shape=[
        jax.ShapeDtypeStruct((2, 2, NN, 64), _F32),
        jax.ShapeDtypeStruct((2, NN, 64), _F32),
    ],
)


def _tc3_body(ap, an, bp, bn, cp0, cp1, cn0, cn1, base2_ref,
              g2_ref, be2_ref, out_ref):
    invp = 1.0 / jnp.maximum(cp0[...] + cp1[...], 1.0)
    invn = 1.0 / jnp.maximum(cn0[...] + cn1[...], 1.0)
    op = ap[...] * invp + an[...] * invn + base2_ref[0]
    on = bp[...] * invp + bn[...] * invn + base2_ref[1]
    op = jnp.maximum(op, 0.0)
    on = jnp.maximum(on, 0.0)
    op, on = _ln_halves(op, on, g2_ref, be2_ref)
    out_ref[:, :64] = op
    out_ref[:, 64:] = on


def _acc2_specs():
    return [
        pl.BlockSpec((BN, 64), lambda i: (i, 0)),
        pl.BlockSpec((BN, 64), lambda i: (i + NEG_BLKS, 0)),
        pl.BlockSpec((BN, 64), lambda i: (i + CORE_BLKS, 0)),
        pl.BlockSpec((BN, 64), lambda i: (i + CORE_BLKS + NEG_BLKS, 0)),
    ]


_tc3 = pl.pallas_call(
    _tc3_body,
    grid=(GRID_N,),
    in_specs=_acc2_specs() + _pcnt_specs() + [
        pl.BlockSpec((2, BN, 64), lambda i: (0, i, 0)),
        _full((128,)), _full((128,)),
    ],
    out_specs=pl.BlockSpec((BN, 128), lambda i: (i, 0)),
    out_shape=jax.ShapeDtypeStruct((NN, 128), _F32),
)


def kernel(x, edge_index, edge_weight, W1p, b1p, W1n, b1n,
           W2p, b2p, W2n, b2n, g1, be1, g2, be2):
    src = edge_index[0]
    dst = edge_index[1]
    zrows = jnp.zeros((R, 64), _F32)
    zcnt = jnp.zeros((R,), _F32)

    tab1, base1 = _tc1(x, W1p, W1n, b1p, b1n)
    psum, pcnt = _sc1(src, dst, edge_weight,
                      tab1.reshape(2 * NN, 64), zrows, zcnt)
    pcnt2 = pcnt.reshape(NC * R, 1)
    t2, base2 = _tc2(psum, psum, psum, psum, pcnt2, pcnt2, pcnt2, pcnt2,
                     base1, W2p, W2n, b2p, b2n, g1, be1)
    acc2 = _sc2(t2.reshape(4 * NN, 64), src, dst, edge_weight, zrows)[0]
    out = _tc3(acc2, acc2, acc2, acc2, pcnt2, pcnt2, pcnt2, pcnt2,
               base2, g2, be2)
    return out
